# baseline (speedup 1.0000x reference)
import functools

import jax
import jax.numpy as jnp
from jax import lax
from jax.experimental import pallas as pl
from jax.experimental.pallas import tpu as pltpu
from jax.experimental.pallas import tpu_sc as plsc

B = 16384
V = 100000
D = 317
H1, H2, H3 = 1024, 512, 256
OUT_D = D + H3


_NC = 2
_NS = 16
_NW = _NC * _NS
_B_PER_W = B // _NW
_K = 16
_NG = _B_PER_W // _K

_sc_mesh = plsc.VectorSubcoreMesh(core_axis_name="c", subcore_axis_name="s")


@functools.partial(
    pl.kernel,
    mesh=_sc_mesh,
    out_type=jax.ShapeDtypeStruct((B, D), jnp.float32),
    scratch_types=[
        pltpu.VMEM((_B_PER_W,), jnp.int32),
        pltpu.SemaphoreType.DMA,
    ],
)
def _sc_gather(table_hbm, idx_hbm, out_hbm, idx_v, sem):
    wid = lax.axis_index("s") * _NC + lax.axis_index("c")
    base = wid * _B_PER_W
    pltpu.sync_copy(idx_hbm.at[pl.ds(base, _B_PER_W)], idx_v)

    def fire(g):
        vec = idx_v[pl.ds(g * _K, _K)]
        for j in range(_K):
            pltpu.async_copy(table_hbm.at[vec[j]], out_hbm.at[base + g * _K + j],
                             sem)

    def drain():
        for _ in range(_K):
            pltpu.make_async_copy(table_hbm.at[0], out_hbm.at[base], sem).wait()

    fire(0)

    def body(g, carry):
        fire(g)
        drain()
        return carry

    lax.fori_loop(1, _NG, body, 0)
    drain()


_TB = 1024


def _mlp_body(feats_ref, emb_ref, w1_ref, b1_ref, w2_ref, b2_ref, w3_ref,
              b3_ref, out_ref):
    f = feats_ref[...]
    h = jnp.dot(f, w1_ref[...], preferred_element_type=jnp.float32)
    h = jnp.maximum(h + b1_ref[...], 0.0)
    h = jnp.dot(h, w2_ref[...], preferred_element_type=jnp.float32)
    h = jnp.maximum(h + b2_ref[...], 0.0)
    z = jnp.dot(h, w3_ref[...], preferred_element_type=jnp.float32)
    z = z + b3_ref[...]
    z = z - jnp.max(z, axis=-1, keepdims=True)
    e = jnp.exp(z)
    sm = e / jnp.sum(e, axis=-1, keepdims=True)
    out_ref[:, :D] = emb_ref[...]
    out_ref[:, D:] = sm


_mlp_call = pl.pallas_call(
    _mlp_body,
    grid=(B // _TB,),
    in_specs=[
        pl.BlockSpec((_TB, 8), lambda i: (i, 0)),
        pl.BlockSpec((_TB, D), lambda i: (i, 0)),
        pl.BlockSpec((8, H1), lambda i: (0, 0)),
        pl.BlockSpec((1, H1), lambda i: (0, 0)),
        pl.BlockSpec((H1, H2), lambda i: (0, 0)),
        pl.BlockSpec((1, H2), lambda i: (0, 0)),
        pl.BlockSpec((H2, H3), lambda i: (0, 0)),
        pl.BlockSpec((1, H3), lambda i: (0, 0)),
    ],
    out_specs=pl.BlockSpec((_TB, OUT_D), lambda i: (i, 0)),
    out_shape=jax.ShapeDtypeStruct((B, OUT_D), jnp.float32),
    compiler_params=pltpu.CompilerParams(
        dimension_semantics=("arbitrary",),
    ),
)


def kernel(anime_id, genre, type, episodes, general_rating, members, user_id,
           user_rating, table, W1, b1, W2, b2, W3, b3):
    idx = genre.astype(jnp.int32)
    emb = _sc_gather(table, idx)

    feats = jnp.stack(
        [anime_id, type, episodes, general_rating, members, user_id,
         user_rating], axis=-1)
    feats = jnp.pad(feats, ((0, 0), (0, 1)))
    w1p = jnp.pad(W1, ((0, 1), (0, 0)))

    out = _mlp_call(feats, emb, w1p, b1.reshape(1, H1), W2,
                    b2.reshape(1, H2), W3, b3.reshape(1, H3))
    return out


# speedup vs baseline: 1.3180x; 1.3180x over previous
import functools

import jax
import jax.numpy as jnp
from jax import lax
from jax.experimental import pallas as pl
from jax.experimental.pallas import tpu as pltpu
from jax.experimental.pallas import tpu_sc as plsc

B = 16384
V = 100000
D = 317
H1, H2, H3 = 1024, 512, 256
OUT_D = D + H3


_NC = 2
_NS = 16
_NW = _NC * _NS
_B_PER_W = B // _NW
_CHUNK = 128
_N_CHUNKS = _B_PER_W // _CHUNK
_DP = 384

_sc_mesh = plsc.VectorSubcoreMesh(core_axis_name="c", subcore_axis_name="s")


@functools.partial(
    pl.kernel,
    mesh=_sc_mesh,
    out_type=jax.ShapeDtypeStruct((B, _DP), jnp.float32),
    scratch_types=[
        pltpu.VMEM((_B_PER_W,), jnp.int32),
        pltpu.VMEM((2, _CHUNK, _DP), jnp.float32),
        pltpu.SemaphoreType.DMA,
        pltpu.SemaphoreType.DMA,
    ],
)
def _sc_gather(table_hbm, idx_hbm, out_hbm, idx_v, rows_v, gsem, wsem):
    wid = lax.axis_index("s") * _NC + lax.axis_index("c")
    base = wid * _B_PER_W
    pltpu.sync_copy(idx_hbm.at[pl.ds(base, _B_PER_W)], idx_v)

    def gather(c, slot):
        pltpu.async_copy(
            table_hbm.at[idx_v.at[pl.ds(c * _CHUNK, _CHUNK)]],
            rows_v.at[slot], gsem)

    def put(c, slot):
        pltpu.async_copy(rows_v.at[slot],
                         out_hbm.at[pl.ds(base + c * _CHUNK, _CHUNK)], wsem)

    gather(0, 0)
    for c in range(_N_CHUNKS):
        slot = c % 2
        pltpu.make_async_copy(table_hbm, rows_v.at[slot], gsem).wait()
        if c >= 1:
            pltpu.make_async_copy(rows_v.at[0], out_hbm.at[pl.ds(0, _CHUNK)],
                                  wsem).wait()
        if c + 1 < _N_CHUNKS:
            gather(c + 1, 1 - slot)
        put(c, slot)
    pltpu.make_async_copy(rows_v.at[0], out_hbm.at[pl.ds(0, _CHUNK)],
                          wsem).wait()


_TB = 1024


def _mlp_body(feats_ref, emb_ref, w1_ref, b1_ref, w2_ref, b2_ref, w3_ref,
              b3_ref, out_ref):
    f = feats_ref[...]
    h = jnp.dot(f, w1_ref[...], preferred_element_type=jnp.float32)
    h = jnp.maximum(h + b1_ref[...], 0.0)
    h = jnp.dot(h, w2_ref[...], preferred_element_type=jnp.float32)
    h = jnp.maximum(h + b2_ref[...], 0.0)
    z = jnp.dot(h, w3_ref[...], preferred_element_type=jnp.float32)
    z = z + b3_ref[...]
    z = z - jnp.max(z, axis=-1, keepdims=True)
    e = jnp.exp(z)
    sm = e / jnp.sum(e, axis=-1, keepdims=True)
    out_ref[:, :D] = emb_ref[:, :D]
    out_ref[:, D:] = sm


_mlp_call = pl.pallas_call(
    _mlp_body,
    grid=(B // _TB,),
    in_specs=[
        pl.BlockSpec((_TB, 8), lambda i: (i, 0)),
        pl.BlockSpec((_TB, _DP), lambda i: (i, 0)),
        pl.BlockSpec((8, H1), lambda i: (0, 0)),
        pl.BlockSpec((1, H1), lambda i: (0, 0)),
        pl.BlockSpec((H1, H2), lambda i: (0, 0)),
        pl.BlockSpec((1, H2), lambda i: (0, 0)),
        pl.BlockSpec((H2, H3), lambda i: (0, 0)),
        pl.BlockSpec((1, H3), lambda i: (0, 0)),
    ],
    out_specs=pl.BlockSpec((_TB, OUT_D), lambda i: (i, 0)),
    out_shape=jax.ShapeDtypeStruct((B, OUT_D), jnp.float32),
    compiler_params=pltpu.CompilerParams(
        dimension_semantics=("arbitrary",),
    ),
)


def kernel(anime_id, genre, type, episodes, general_rating, members, user_id,
           user_rating, table, W1, b1, W2, b2, W3, b3):
    idx = genre.astype(jnp.int32)
    table_p = jnp.pad(table, ((0, 0), (0, _DP - D)))
    emb = _sc_gather(table_p, idx)

    feats = jnp.stack(
        [anime_id, type, episodes, general_rating, members, user_id,
         user_rating], axis=-1)
    feats = jnp.pad(feats, ((0, 0), (0, 1)))
    w1p = jnp.pad(W1, ((0, 1), (0, 0)))

    out = _mlp_call(feats, emb, w1p, b1.reshape(1, H1), W2,
                    b2.reshape(1, H2), W3, b3.reshape(1, H3))
    return out


# speedup vs baseline: 2.7963x; 2.1217x over previous
import functools

import jax
import jax.numpy as jnp
from jax import lax
from jax.experimental import pallas as pl
from jax.experimental.pallas import tpu as pltpu
from jax.experimental.pallas import tpu_sc as plsc

B = 16384
V = 100000
D = 317
H1, H2, H3 = 1024, 512, 256
OUT_D = D + H3


_NC = 2
_NS = 16
_NW = _NC * _NS
_B_PER_W = B // _NW
_CHUNK = 128
_N_CHUNKS = _B_PER_W // _CHUNK
_DP = 384

_sc_mesh = plsc.VectorSubcoreMesh(core_axis_name="c", subcore_axis_name="s")


@functools.partial(
    pl.kernel,
    mesh=_sc_mesh,
    out_type=jax.ShapeDtypeStruct((B, _DP), jnp.float32),
    scratch_types=[
        pltpu.VMEM((_B_PER_W,), jnp.int32),
        pltpu.VMEM((2, _CHUNK, _DP), jnp.float32),
        pltpu.SemaphoreType.DMA,
        pltpu.SemaphoreType.DMA,
    ],
)
def _sc_gather(table_hbm, tail_hbm, idx_hbm, out_hbm, idx_v, rows_v, gsem, wsem):
    wid = lax.axis_index("s") * _NC + lax.axis_index("c")
    base = wid * _B_PER_W
    pltpu.sync_copy(idx_hbm.at[pl.ds(base, _B_PER_W)], idx_v)

    def gather(c, slot):
        ids = idx_v.at[pl.ds(c * _CHUNK, _CHUNK)]
        pltpu.async_copy(table_hbm.at[ids, pl.ds(0, 128)],
                         rows_v.at[slot, :, pl.ds(0, 128)], gsem)
        pltpu.async_copy(table_hbm.at[ids, pl.ds(128, 128)],
                         rows_v.at[slot, :, pl.ds(128, 128)], gsem)
        pltpu.async_copy(tail_hbm.at[ids],
                         rows_v.at[slot, :, pl.ds(256, 128)], gsem)

    def put(c, slot):
        pltpu.async_copy(rows_v.at[slot],
                         out_hbm.at[pl.ds(base + c * _CHUNK, _CHUNK)], wsem)

    gather(0, 0)
    for c in range(_N_CHUNKS):
        slot = c % 2
        pltpu.make_async_copy(table_hbm, rows_v.at[slot], gsem).wait()
        if c >= 1:
            pltpu.make_async_copy(rows_v.at[0], out_hbm.at[pl.ds(0, _CHUNK)],
                                  wsem).wait()
        if c + 1 < _N_CHUNKS:
            gather(c + 1, 1 - slot)
        put(c, slot)
    pltpu.make_async_copy(rows_v.at[0], out_hbm.at[pl.ds(0, _CHUNK)],
                          wsem).wait()


_RB = 2000
_TAIL = D - 256


def _tail_body(t_ref, out_ref, vacc, sem):
    i = pl.program_id(0)
    cp = pltpu.make_async_copy(
        t_ref.at[pl.ds(i * _RB, _RB), pl.ds(256, _TAIL)], vacc, sem)
    cp.start()
    out_ref[:, _TAIL:] = jnp.zeros((_RB, 128 - _TAIL), jnp.float32)
    cp.wait()
    out_ref[:, :_TAIL] = vacc[...]


_tail_call = pl.pallas_call(
    _tail_body,
    grid=(V // _RB,),
    in_specs=[pl.BlockSpec(memory_space=pl.ANY)],
    out_specs=pl.BlockSpec((_RB, 128), lambda i: (i, 0)),
    out_shape=jax.ShapeDtypeStruct((V, 128), jnp.float32),
    scratch_shapes=[pltpu.VMEM((_RB, _TAIL), jnp.float32),
                    pltpu.SemaphoreType.DMA],
    compiler_params=pltpu.CompilerParams(
        dimension_semantics=("arbitrary",),
    ),
)


_TB = 1024


def _mlp_body(feats_ref, emb_ref, w1_ref, b1_ref, w2_ref, b2_ref, w3_ref,
              b3_ref, out_ref):
    f = feats_ref[...]
    h = jnp.dot(f, w1_ref[...], preferred_element_type=jnp.float32)
    h = jnp.maximum(h + b1_ref[...], 0.0)
    h = jnp.dot(h, w2_ref[...], preferred_element_type=jnp.float32)
    h = jnp.maximum(h + b2_ref[...], 0.0)
    z = jnp.dot(h, w3_ref[...], preferred_element_type=jnp.float32)
    z = z + b3_ref[...]
    z = z - jnp.max(z, axis=-1, keepdims=True)
    e = jnp.exp(z)
    sm = e / jnp.sum(e, axis=-1, keepdims=True)
    out_ref[:, :D] = emb_ref[:, :D]
    out_ref[:, D:] = sm


_mlp_call = pl.pallas_call(
    _mlp_body,
    grid=(B // _TB,),
    in_specs=[
        pl.BlockSpec((_TB, 8), lambda i: (i, 0)),
        pl.BlockSpec((_TB, _DP), lambda i: (i, 0)),
        pl.BlockSpec((8, H1), lambda i: (0, 0)),
        pl.BlockSpec((1, H1), lambda i: (0, 0)),
        pl.BlockSpec((H1, H2), lambda i: (0, 0)),
        pl.BlockSpec((1, H2), lambda i: (0, 0)),
        pl.BlockSpec((H2, H3), lambda i: (0, 0)),
        pl.BlockSpec((1, H3), lambda i: (0, 0)),
    ],
    out_specs=pl.BlockSpec((_TB, OUT_D), lambda i: (i, 0)),
    out_shape=jax.ShapeDtypeStruct((B, OUT_D), jnp.float32),
    compiler_params=pltpu.CompilerParams(
        dimension_semantics=("arbitrary",),
    ),
)


def kernel(anime_id, genre, type, episodes, general_rating, members, user_id,
           user_rating, table, W1, b1, W2, b2, W3, b3):
    idx = genre.astype(jnp.int32)
    tail = _tail_call(table)
    emb = _sc_gather(table, tail, idx)

    feats = jnp.stack(
        [anime_id, type, episodes, general_rating, members, user_id,
         user_rating], axis=-1)
    feats = jnp.pad(feats, ((0, 0), (0, 1)))
    w1p = jnp.pad(W1, ((0, 1), (0, 0)))

    out = _mlp_call(feats, emb, w1p, b1.reshape(1, H1), W2,
                    b2.reshape(1, H2), W3, b3.reshape(1, H3))
    return out
